# baseline (speedup 1.0000x reference)
import functools

import jax
import jax.numpy as jnp
from jax import lax
from jax.experimental import pallas as pl
from jax.experimental.pallas import tpu as pltpu
from jax.experimental.pallas import tpu_sc as plsc

_N = 10000
_H = 128
_EPS = 1e-05

_NC = 2
_NS = 16
_NW = _NC * _NS
_K = 128
_CHUNKS = 80
_EPT = _CHUNKS * _K
_E_PAD = _NW * _EPT
_N_PAD = 10240
_RPT = _N_PAD // _NS

_sc_mesh = plsc.VectorSubcoreMesh(core_axis_name="c", subcore_axis_name="s")
_sc_params = pltpu.CompilerParams(needs_layout_passes=False)


def _deg_body(row_hbm, ones_hbm, zeros_hbm, out_hbm, ridx_v, ones_v, deg_sh):
    c = lax.axis_index("c")
    s = lax.axis_index("s")
    wid = s * _NC + c
    pltpu.sync_copy(row_hbm.at[wid], ridx_v)
    pltpu.sync_copy(ones_hbm, ones_v)
    pltpu.sync_copy(zeros_hbm, deg_sh.at[pl.ds(s * _RPT, _RPT)])
    plsc.subcore_barrier()

    def body(j, carry):
        pltpu.sync_copy(ones_v, deg_sh.at[ridx_v.at[j]], add=True)
        return carry

    lax.fori_loop(0, _CHUNKS, body, 0)
    plsc.subcore_barrier()
    pltpu.sync_copy(deg_sh.at[pl.ds(s * _RPT, _RPT)],
                    out_hbm.at[c, pl.ds(s * _RPT, _RPT)])


_deg_call = functools.partial(
    pl.kernel,
    out_type=jax.ShapeDtypeStruct((_NC, _N_PAD), jnp.float32),
    mesh=_sc_mesh,
    scratch_types=[
        pltpu.VMEM((_CHUNKS, _K), jnp.int32),
        pltpu.VMEM((_K,), jnp.float32),
        pltpu.VMEM_SHARED((_N_PAD,), jnp.float32),
    ],
)(_deg_body)


_IBLK = 8
_NBLK = _CHUNKS // _IBLK
_NBLK_P = _NBLK + 2


def _prop_body(hs_hbm, row_hbm, col_hbm, zeros_hbm, out_hbm,
               cidx_v, ri0, ri1, gb0, gb1, agg_sh, sg0, sg1, si0, si1):
    c = lax.axis_index("c")
    s = lax.axis_index("s")
    wid = s * _NC + c
    pltpu.sync_copy(col_hbm.at[wid], cidx_v)
    pltpu.sync_copy(zeros_hbm, agg_sh.at[pl.ds(s * _RPT, _RPT)])
    plsc.subcore_barrier()

    ribufs = (ri0, ri1)
    isems = (si0, si1)
    gbufs = (gb0, gb1)
    gsems = (sg0, sg1)

    def fire_idx(blk, m):
        pltpu.async_copy(row_hbm.at[wid, blk], ribufs[m], isems[m])

    def drain_idx(m):
        pltpu.make_async_copy(row_hbm.at[0, 0], ribufs[m], isems[m]).wait()

    def fire_g(m, k, gb):
        pltpu.async_copy(hs_hbm.at[ribufs[m].at[k]], gbufs[gb], gsems[gb])

    def drain_g(gb):
        pltpu.make_async_copy(hs_hbm.at[pl.ds(0, _K)], gbufs[gb], gsems[gb]).wait()

    fire_idx(0, 0)
    fire_idx(1, 1)
    drain_idx(0)
    fire_g(0, 0, 0)
    fire_g(0, 1, 1)

    def section(p, ib):
        base = 16 * p + 8 * ib
        for k in range(_IBLK):
            gb = k % 2
            drain_g(gb)
            pltpu.sync_copy(gbufs[gb], agg_sh.at[cidx_v.at[base + k]], add=True)
            if k < _IBLK - 2:
                fire_g(ib, k + 2, gb)
            elif k == _IBLK - 2:
                drain_idx(1 - ib)
                fire_g(1 - ib, 0, gb)
            else:
                fire_g(1 - ib, 1, gb)
        fire_idx_blk = 2 * p + ib + 2
        pltpu.async_copy(row_hbm.at[wid, fire_idx_blk], ribufs[ib], isems[ib])

    def body(p, carry):
        section(p, 0)
        section(p, 1)
        return carry

    lax.fori_loop(0, _NBLK // 2, body, 0)
    drain_g(0)
    drain_g(1)
    drain_idx(1)
    plsc.subcore_barrier()
    pltpu.sync_copy(agg_sh.at[pl.ds(s * _RPT, _RPT)],
                    out_hbm.at[c, pl.ds(s * _RPT, _RPT)])


_prop_call = functools.partial(
    pl.kernel,
    out_type=jax.ShapeDtypeStruct((_NC, _N_PAD, _H), jnp.float32),
    mesh=_sc_mesh,
    scratch_types=[
        pltpu.VMEM((_CHUNKS, _K), jnp.int32),
        pltpu.VMEM((_IBLK, _K), jnp.int32),
        pltpu.VMEM((_IBLK, _K), jnp.int32),
        pltpu.VMEM((_K, _H), jnp.float32),
        pltpu.VMEM((_K, _H), jnp.float32),
        pltpu.VMEM_SHARED((_N_PAD, _H), jnp.float32),
        pltpu.SemaphoreType.DMA,
        pltpu.SemaphoreType.DMA,
        pltpu.SemaphoreType.DMA,
        pltpu.SemaphoreType.DMA,
    ],
)(_prop_body)


_KS = 64
_SCHUNKS = _EPT // _KS
_SCHUNKS_P = _SCHUNKS + 2
_SBUF = _SCHUNKS_P * _KS


def _score_body(x_hbm, row_hbm, col_hbm, out_hbm,
                ridx_v, cidx_v, rb0, cb0, rb1, cb1, sbuf, pbuf,
                sr0, sc0, sr1, sc1):
    c = lax.axis_index("c")
    s = lax.axis_index("s")
    wid = s * _NC + c
    pltpu.sync_copy(row_hbm.at[wid], ridx_v)
    pltpu.sync_copy(col_hbm.at[wid], cidx_v)

    rbufs = (rb0, rb1)
    cbufs = (cb0, cb1)
    rsems = (sr0, sr1)
    csems = (sc0, sc1)
    lane = lax.iota(jnp.int32, 16)

    def fire(j, b):
        pltpu.async_copy(x_hbm.at[ridx_v.at[j]], rbufs[b], rsems[b])
        pltpu.async_copy(x_hbm.at[cidx_v.at[j]], cbufs[b], csems[b])

    def drain(b):
        pltpu.make_async_copy(x_hbm.at[pl.ds(0, _KS)], rbufs[b], rsems[b]).wait()
        pltpu.make_async_copy(x_hbm.at[pl.ds(0, _KS)], cbufs[b], csems[b]).wait()

    def compute(j, b):
        rb = rbufs[b]
        cb = cbufs[b]
        for g in range(_KS // 16):
            for e16 in range(16):
                e = g * 16 + e16
                acc = rb[e, pl.ds(0, 16)] * cb[e, pl.ds(0, 16)]
                for q in range(1, 8):
                    acc = acc + rb[e, pl.ds(q * 16, 16)] * cb[e, pl.ds(q * 16, 16)]
                pbuf[e16, :] = acc
            tot = plsc.load_gather(pbuf, [lane, jnp.zeros((16,), jnp.int32)])
            for q in range(1, 16):
                tot = tot + plsc.load_gather(pbuf, [lane, jnp.full((16,), q, jnp.int32)])
            sig = 1.0 / (1.0 + jnp.exp(-tot))
            sbuf[pl.ds(j * _KS + g * 16, 16)] = sig

    for b in range(2):
        fire(b, b)

    def body(i, carry):
        for b in range(2):
            j = 2 * i + b
            drain(b)
            compute(j, b)
            fire(j + 2, b)
        return carry

    lax.fori_loop(0, _SCHUNKS // 2, body, 0)
    drain(0)
    drain(1)
    pltpu.sync_copy(sbuf.at[pl.ds(0, _EPT)], out_hbm.at[pl.ds(wid * _EPT, _EPT)])


_score_call = functools.partial(
    pl.kernel,
    out_type=jax.ShapeDtypeStruct((_E_PAD,), jnp.float32),
    mesh=_sc_mesh,
    scratch_types=[
        pltpu.VMEM((_SCHUNKS_P, _KS), jnp.int32),
        pltpu.VMEM((_SCHUNKS_P, _KS), jnp.int32),
        pltpu.VMEM((_KS, _H), jnp.float32),
        pltpu.VMEM((_KS, _H), jnp.float32),
        pltpu.VMEM((_KS, _H), jnp.float32),
        pltpu.VMEM((_KS, _H), jnp.float32),
        pltpu.VMEM((_SBUF,), jnp.float32),
        pltpu.VMEM((16, 16), jnp.float32),
        pltpu.SemaphoreType.DMA,
        pltpu.SemaphoreType.DMA,
        pltpu.SemaphoreType.DMA,
        pltpu.SemaphoreType.DMA,
    ],
    compiler_params=_sc_params,
)(_score_body)


def _bn(x, g, b):
    m = jnp.mean(x, axis=0, keepdims=True)
    v = jnp.mean((x - m) ** 2, axis=0, keepdims=True)
    return (x - m) * lax.rsqrt(v + _EPS) * g + b


def _tc0_body(x_ref, gf_ref, bfeat_ref, wf_ref, bf_ref, degs_ref,
              g0_ref, b0_ref, wc0_ref, hs_ref, dis_ref):
    xb = _bn(x_ref[...], gf_ref[...], bfeat_ref[...])
    x1 = jnp.maximum(
        jnp.dot(xb, wf_ref[...], preferred_element_type=jnp.float32) + bf_ref[...],
        0.0)
    deg = degs_ref[0] + degs_ref[1] + 1.0
    dis = lax.rsqrt(deg)[: _N]
    xb1 = _bn(x1, g0_ref[...], b0_ref[...])
    hs_ref[...] = dis * jnp.dot(xb1, wc0_ref[...], preferred_element_type=jnp.float32)
    dis_ref[...] = dis


_tc0_call = pl.pallas_call(
    _tc0_body,
    out_shape=[
        jax.ShapeDtypeStruct((_N, _H), jnp.float32),
        jax.ShapeDtypeStruct((_N, 1), jnp.float32),
    ],
)


def _tc_mid_body(agg_ref, hs_ref, dis_ref, bc_ref, g_ref, b_ref, wc_ref, out_ref):
    agg = agg_ref[0, : _N] + agg_ref[1, : _N]
    x2 = jnp.maximum(dis_ref[...] * (agg + hs_ref[...]) + bc_ref[...], 0.0)
    xb = _bn(x2, g_ref[...], b_ref[...])
    out_ref[...] = dis_ref[...] * jnp.dot(
        xb, wc_ref[...], preferred_element_type=jnp.float32)


_tc_mid_call = pl.pallas_call(
    _tc_mid_body,
    out_shape=jax.ShapeDtypeStruct((_N, _H), jnp.float32),
)


def _tc_last_body(agg_ref, hs_ref, dis_ref, bc_ref, out_ref):
    agg = agg_ref[0, : _N] + agg_ref[1, : _N]
    out_ref[...] = dis_ref[...] * (agg + hs_ref[...]) + bc_ref[...]


_tc_last_call = pl.pallas_call(
    _tc_last_body,
    out_shape=jax.ShapeDtypeStruct((_N, _H), jnp.float32),
)


def kernel(x, edge_index, bn_feat_g, bn_feat_b, Wf, bf, bn_g, bn_b, Wc, bc):
    row = edge_index[0]
    col = edge_index[1]
    e = row.shape[0]
    pad = _E_PAD - e
    zpad = jnp.zeros((pad,), jnp.int32)
    tpad = jnp.full((pad,), _N, jnp.int32)
    row0f = jnp.concatenate([row, zpad])
    col0f = jnp.concatenate([col, zpad])
    row0p = jnp.concatenate(
        [row0f.reshape(_NW, _NBLK, _IBLK, _K),
         jnp.zeros((_NW, 2, _IBLK, _K), jnp.int32)], axis=1)
    rowN = jnp.concatenate([row, tpad]).reshape(_NW, _CHUNKS, _K)
    colN = jnp.concatenate([col, tpad]).reshape(_NW, _CHUNKS, _K)
    spad = jnp.zeros((_NW, 2 * _KS), jnp.int32)
    row0s = jnp.concatenate(
        [row0f.reshape(_NW, _EPT), spad], axis=1).reshape(_NW, _SCHUNKS_P, _KS)
    col0s = jnp.concatenate(
        [col0f.reshape(_NW, _EPT), spad], axis=1).reshape(_NW, _SCHUNKS_P, _KS)
    zeros1 = jnp.zeros((_RPT,), jnp.float32)
    zeros2 = jnp.zeros((_RPT, _H), jnp.float32)
    ones_k = jnp.ones((_K,), jnp.float32)

    gf = bn_feat_g.reshape(1, _H)
    bfeat = bn_feat_b.reshape(1, _H)
    bf2 = bf.reshape(1, _H)

    degs = _deg_call(rowN, ones_k, zeros1)
    hs0, dis = _tc0_call(x, gf, bfeat, Wf, bf2, degs.reshape(_NC, _N_PAD, 1),
                         bn_g[0].reshape(1, _H), bn_b[0].reshape(1, _H), Wc[0])
    agg0 = _prop_call(hs0, row0p, colN, zeros2)
    hs1 = _tc_mid_call(agg0, hs0, dis, bc[0].reshape(1, _H),
                       bn_g[1].reshape(1, _H), bn_b[1].reshape(1, _H), Wc[1])
    agg1 = _prop_call(hs1, row0p, colN, zeros2)
    hs2 = _tc_mid_call(agg1, hs1, dis, bc[1].reshape(1, _H),
                       bn_g[2].reshape(1, _H), bn_b[2].reshape(1, _H), Wc[2])
    agg2 = _prop_call(hs2, row0p, colN, zeros2)
    x4 = _tc_last_call(agg2, hs2, dis, bc[2].reshape(1, _H))
    scores = _score_call(x4, row0s, col0s)
    return scores[:e]


# speedup vs baseline: 1.8188x; 1.8188x over previous
import functools

import jax
import jax.numpy as jnp
from jax import lax
from jax.experimental import pallas as pl
from jax.experimental.pallas import tpu as pltpu
from jax.experimental.pallas import tpu_sc as plsc

_N = 10000
_H = 128
_EPS = 1e-05

_NC = 2
_NS = 16
_NW = _NC * _NS
_K = 128
_CHUNKS = 80
_EPT = _CHUNKS * _K
_E_PAD = _NW * _EPT
_N_PAD = 10240
_RPT = _N_PAD // _NS

_sc_mesh = plsc.VectorSubcoreMesh(core_axis_name="c", subcore_axis_name="s")
_sc_params = pltpu.CompilerParams(needs_layout_passes=False)


def _deg_body(row_hbm, ones_hbm, zeros_hbm, out_hbm, ridx_v, ones_v, deg_sh):
    c = lax.axis_index("c")
    s = lax.axis_index("s")
    wid = s * _NC + c
    pltpu.sync_copy(row_hbm.at[wid], ridx_v)
    pltpu.sync_copy(ones_hbm, ones_v)
    pltpu.sync_copy(zeros_hbm, deg_sh.at[pl.ds(s * _RPT, _RPT)])
    plsc.subcore_barrier()

    def body(j, carry):
        pltpu.sync_copy(ones_v, deg_sh.at[ridx_v.at[j]], add=True)
        return carry

    lax.fori_loop(0, _CHUNKS, body, 0)
    plsc.subcore_barrier()
    pltpu.sync_copy(deg_sh.at[pl.ds(s * _RPT, _RPT)],
                    out_hbm.at[c, pl.ds(s * _RPT, _RPT)])


_deg_call = functools.partial(
    pl.kernel,
    out_type=jax.ShapeDtypeStruct((_NC, _N_PAD), jnp.float32),
    mesh=_sc_mesh,
    scratch_types=[
        pltpu.VMEM((_CHUNKS, _K), jnp.int32),
        pltpu.VMEM((_K,), jnp.float32),
        pltpu.VMEM_SHARED((_N_PAD,), jnp.float32),
    ],
)(_deg_body)


_N_STAGE = 10112
_SLAB = _N_STAGE // _NS


def _prop_body(hs_hbm, row_hbm, col_hbm, zeros_hbm, out_hbm,
               ridx_v, cidx_v, gbuf, agg_sh, sem):
    c = lax.axis_index("c")
    s = lax.axis_index("s")
    wid = s * _NC + c
    pltpu.sync_copy(row_hbm.at[wid], ridx_v)
    pltpu.sync_copy(col_hbm.at[wid], cidx_v)
    pltpu.sync_copy(zeros_hbm, agg_sh.at[pl.ds(s * _RPT, _RPT)])
    plsc.subcore_barrier()

    def body(j, carry):
        pltpu.async_copy(hs_hbm.at[ridx_v.at[j]], gbuf, sem).wait()
        pltpu.sync_copy(gbuf, agg_sh.at[cidx_v.at[j]], add=True)
        return carry

    lax.fori_loop(0, _CHUNKS, body, 0)
    plsc.subcore_barrier()
    pltpu.sync_copy(agg_sh.at[pl.ds(s * _RPT, _RPT)],
                    out_hbm.at[c, pl.ds(s * _RPT, _RPT)])


_prop_call = functools.partial(
    pl.kernel,
    out_type=jax.ShapeDtypeStruct((_NC, _N_PAD, _H), jnp.float32),
    mesh=_sc_mesh,
    scratch_types=[
        pltpu.VMEM((_CHUNKS, _K), jnp.int32),
        pltpu.VMEM((_CHUNKS, _K), jnp.int32),
        pltpu.VMEM((_K, _H), jnp.float32),
        pltpu.VMEM_SHARED((_N_PAD, _H), jnp.float32),
        pltpu.SemaphoreType.DMA,
    ],
)(_prop_body)


_KS = 64
_SCHUNKS = _EPT // _KS


def _score_body(x_hbm, row_hbm, col_hbm, out_hbm,
                ridx_v, cidx_v, rbuf, cbuf, sbuf, pbuf, x_sh):
    c = lax.axis_index("c")
    s = lax.axis_index("s")
    wid = s * _NC + c
    pltpu.sync_copy(row_hbm.at[wid], ridx_v)
    pltpu.sync_copy(col_hbm.at[wid], cidx_v)
    pltpu.sync_copy(x_hbm.at[pl.ds(s * _SLAB, _SLAB)],
                    x_sh.at[pl.ds(s * _SLAB, _SLAB)])
    plsc.subcore_barrier()

    lane = lax.iota(jnp.int32, 16)

    def group(g, base, rbuf, cbuf):
        e0 = g * 16
        for e16 in range(16):
            e = e0 + e16
            acc = rbuf[e, pl.ds(0, 16)] * cbuf[e, pl.ds(0, 16)]
            for q in range(1, 8):
                acc = acc + rbuf[e, pl.ds(q * 16, 16)] * cbuf[e, pl.ds(q * 16, 16)]
            pbuf[e16, :] = acc
        cols = jnp.zeros((16,), jnp.int32)
        tot = plsc.load_gather(pbuf, [lane, cols])
        for q in range(1, 16):
            cols = cols + 1
            tot = tot + plsc.load_gather(pbuf, [lane, cols])
        sig = 1.0 / (1.0 + jnp.exp(-tot))
        sbuf[pl.ds(base + g * 16, 16)] = sig

    def body(jj, carry):
        for h in range(2):
            pltpu.sync_copy(x_sh.at[ridx_v.at[jj, pl.ds(h * _KS, _KS)]], rbuf)
            pltpu.sync_copy(x_sh.at[cidx_v.at[jj, pl.ds(h * _KS, _KS)]], cbuf)
            base = jj * _K + h * _KS

            def gbody(g, carry2):
                group(g, base, rbuf, cbuf)
                return carry2

            lax.fori_loop(0, _KS // 16, gbody, 0)
        return carry

    lax.fori_loop(0, _CHUNKS, body, 0)
    pltpu.sync_copy(sbuf, out_hbm.at[pl.ds(wid * _EPT, _EPT)])


_score_call = functools.partial(
    pl.kernel,
    out_type=jax.ShapeDtypeStruct((_E_PAD,), jnp.float32),
    mesh=_sc_mesh,
    scratch_types=[
        pltpu.VMEM((_CHUNKS, _K), jnp.int32),
        pltpu.VMEM((_CHUNKS, _K), jnp.int32),
        pltpu.VMEM((_KS, _H), jnp.float32),
        pltpu.VMEM((_KS, _H), jnp.float32),
        pltpu.VMEM((_EPT,), jnp.float32),
        pltpu.VMEM((16, 16), jnp.float32),
        pltpu.VMEM_SHARED((_N_STAGE, _H), jnp.float32),
    ],
    compiler_params=_sc_params,
)(_score_body)


def _bn(x, g, b):
    m = jnp.mean(x, axis=0, keepdims=True)
    v = jnp.mean((x - m) ** 2, axis=0, keepdims=True)
    return (x - m) * lax.rsqrt(v + _EPS) * g + b


def _tc0_body(x_ref, gf_ref, bfeat_ref, wf_ref, bf_ref, degs_ref,
              g0_ref, b0_ref, wc0_ref, hs_ref, dis_ref):
    xb = _bn(x_ref[...], gf_ref[...], bfeat_ref[...])
    x1 = jnp.maximum(
        jnp.dot(xb, wf_ref[...], preferred_element_type=jnp.float32) + bf_ref[...],
        0.0)
    deg = degs_ref[0] + degs_ref[1] + 1.0
    dis = lax.rsqrt(deg)[: _N]
    xb1 = _bn(x1, g0_ref[...], b0_ref[...])
    hs_ref[...] = dis * jnp.dot(xb1, wc0_ref[...],
                                preferred_element_type=jnp.float32)
    dis_ref[...] = dis


_tc0_call = pl.pallas_call(
    _tc0_body,
    out_shape=[
        jax.ShapeDtypeStruct((_N, _H), jnp.float32),
        jax.ShapeDtypeStruct((_N, 1), jnp.float32),
    ],
)


def _tc_mid_body(agg_ref, hs_ref, dis_ref, bc_ref, g_ref, b_ref, wc_ref, out_ref):
    agg = agg_ref[0, : _N] + agg_ref[1, : _N]
    x2 = jnp.maximum(dis_ref[...] * (agg + hs_ref[...]) + bc_ref[...], 0.0)
    xb = _bn(x2, g_ref[...], b_ref[...])
    out_ref[...] = dis_ref[...] * jnp.dot(
        xb, wc_ref[...], preferred_element_type=jnp.float32)


_tc_mid_call = pl.pallas_call(
    _tc_mid_body,
    out_shape=jax.ShapeDtypeStruct((_N, _H), jnp.float32),
)


def _tc_last_body(agg_ref, hs_ref, dis_ref, bc_ref, out_ref):
    agg = agg_ref[0, : _N] + agg_ref[1, : _N]
    out_ref[: _N] = dis_ref[...] * (agg + hs_ref[...]) + bc_ref[...]


_tc_last_call = pl.pallas_call(
    _tc_last_body,
    out_shape=jax.ShapeDtypeStruct((_N_STAGE, _H), jnp.float32),
)


def kernel(x, edge_index, bn_feat_g, bn_feat_b, Wf, bf, bn_g, bn_b, Wc, bc):
    row = edge_index[0]
    col = edge_index[1]
    e = row.shape[0]
    pad = _E_PAD - e
    zpad = jnp.zeros((pad,), jnp.int32)
    tpad = jnp.full((pad,), _N, jnp.int32)
    row0f = jnp.concatenate([row, zpad])
    col0f = jnp.concatenate([col, zpad])
    row0p = row0f.reshape(_NW, _CHUNKS, _K)
    colNp = jnp.concatenate([col, tpad]).reshape(_NW, _CHUNKS, _K)
    rowN = jnp.concatenate([row, tpad]).reshape(_NW, _CHUNKS, _K)
    row0s = row0f.reshape(_NW, _CHUNKS, _K)
    col0s = col0f.reshape(_NW, _CHUNKS, _K)
    zeros1 = jnp.zeros((_RPT,), jnp.float32)
    zeros2 = jnp.zeros((_RPT, _H), jnp.float32)
    ones_k = jnp.ones((_K,), jnp.float32)

    gf = bn_feat_g.reshape(1, _H)
    bfeat = bn_feat_b.reshape(1, _H)
    bf2 = bf.reshape(1, _H)

    degs = _deg_call(rowN, ones_k, zeros1)
    hs0, dis = _tc0_call(x, gf, bfeat, Wf, bf2, degs.reshape(_NC, _N_PAD, 1),
                         bn_g[0].reshape(1, _H), bn_b[0].reshape(1, _H), Wc[0])
    agg0 = _prop_call(hs0, row0p, colNp, zeros2)
    hs1 = _tc_mid_call(agg0, hs0, dis, bc[0].reshape(1, _H),
                       bn_g[1].reshape(1, _H), bn_b[1].reshape(1, _H), Wc[1])
    agg1 = _prop_call(hs1, row0p, colNp, zeros2)
    hs2 = _tc_mid_call(agg1, hs1, dis, bc[1].reshape(1, _H),
                       bn_g[2].reshape(1, _H), bn_b[2].reshape(1, _H), Wc[2])
    agg2 = _prop_call(hs2, row0p, colNp, zeros2)
    x4 = _tc_last_call(agg2, hs2, dis, bc[2].reshape(1, _H))
    scores = _score_call(x4, row0s, col0s)
    return scores[:e]


# speedup vs baseline: 1.8498x; 1.0170x over previous
import functools

import jax
import jax.numpy as jnp
from jax import lax
from jax.experimental import pallas as pl
from jax.experimental.pallas import tpu as pltpu
from jax.experimental.pallas import tpu_sc as plsc

_N = 10000
_H = 128
_EPS = 1e-05

_NC = 2
_NS = 16
_NW = _NC * _NS
_K = 128
_CHUNKS = 80
_EPT = _CHUNKS * _K
_E_PAD = _NW * _EPT
_N_PAD = 10240
_RPT = _N_PAD // _NS

_sc_mesh = plsc.VectorSubcoreMesh(core_axis_name="c", subcore_axis_name="s")
_sc_params = pltpu.CompilerParams(needs_layout_passes=False)


def _deg_body(row_hbm, ones_hbm, zeros_hbm, out_hbm, ridx_v, ones_v, deg_sh):
    c = lax.axis_index("c")
    s = lax.axis_index("s")
    wid = s * _NC + c
    pltpu.sync_copy(row_hbm.at[wid], ridx_v)
    pltpu.sync_copy(ones_hbm, ones_v)
    pltpu.sync_copy(zeros_hbm, deg_sh.at[pl.ds(s * _RPT, _RPT)])
    plsc.subcore_barrier()

    def body(j, carry):
        pltpu.sync_copy(ones_v, deg_sh.at[ridx_v.at[j]], add=True)
        return carry

    lax.fori_loop(0, _CHUNKS, body, 0)
    plsc.subcore_barrier()
    pltpu.sync_copy(deg_sh.at[pl.ds(s * _RPT, _RPT)],
                    out_hbm.at[c, pl.ds(s * _RPT, _RPT)])


_deg_call = functools.partial(
    pl.kernel,
    out_type=jax.ShapeDtypeStruct((_NC, _N_PAD), jnp.float32),
    mesh=_sc_mesh,
    scratch_types=[
        pltpu.VMEM((_CHUNKS, _K), jnp.int32),
        pltpu.VMEM((_K,), jnp.float32),
        pltpu.VMEM_SHARED((_N_PAD,), jnp.float32),
    ],
)(_deg_body)


_N_STAGE = 10112
_SLAB = _N_STAGE // _NS


_PBLK = 16
_PNBLK = _CHUNKS // _PBLK


def _prop_body(hs_hbm, row_hbm, col_hbm, zeros_hbm, out_hbm,
               ribuf, cidx_v, gb0, gb1, agg_sh, s0, s1):
    c = lax.axis_index("c")
    s = lax.axis_index("s")
    wid = s * _NC + c
    pltpu.sync_copy(col_hbm.at[wid], cidx_v)
    pltpu.sync_copy(zeros_hbm, agg_sh.at[pl.ds(s * _RPT, _RPT)])
    plsc.subcore_barrier()

    def body(blk, carry):
        pltpu.sync_copy(row_hbm.at[wid, blk], ribuf)
        for kk in range(_PBLK // 2):
            j = blk * _PBLK + 2 * kk
            pltpu.async_copy(hs_hbm.at[ribuf.at[2 * kk]], gb0, s0)
            pltpu.async_copy(hs_hbm.at[ribuf.at[2 * kk + 1]], gb1, s1)
            pltpu.make_async_copy(hs_hbm.at[pl.ds(0, _K)], gb0, s0).wait()
            pltpu.sync_copy(gb0, agg_sh.at[cidx_v.at[j]], add=True)
            pltpu.make_async_copy(hs_hbm.at[pl.ds(0, _K)], gb1, s1).wait()
            pltpu.sync_copy(gb1, agg_sh.at[cidx_v.at[j + 1]], add=True)
        return carry

    lax.fori_loop(0, _PNBLK, body, 0)
    plsc.subcore_barrier()
    pltpu.sync_copy(agg_sh.at[pl.ds(s * _RPT, _RPT)],
                    out_hbm.at[c, pl.ds(s * _RPT, _RPT)])


_prop_call = functools.partial(
    pl.kernel,
    out_type=jax.ShapeDtypeStruct((_NC, _N_PAD, _H), jnp.float32),
    mesh=_sc_mesh,
    scratch_types=[
        pltpu.VMEM((_PBLK, _K), jnp.int32),
        pltpu.VMEM((_CHUNKS, _K), jnp.int32),
        pltpu.VMEM((_K, _H), jnp.float32),
        pltpu.VMEM((_K, _H), jnp.float32),
        pltpu.VMEM_SHARED((_N_PAD, _H), jnp.float32),
        pltpu.SemaphoreType.DMA,
        pltpu.SemaphoreType.DMA,
    ],
)(_prop_body)


_KS = 64
_SCHUNKS = _EPT // _KS


def _score_body(x_hbm, row_hbm, col_hbm, out_hbm,
                ridx_v, cidx_v, rbuf, cbuf, sbuf, pbuf, x_sh):
    c = lax.axis_index("c")
    s = lax.axis_index("s")
    wid = s * _NC + c
    pltpu.sync_copy(row_hbm.at[wid], ridx_v)
    pltpu.sync_copy(col_hbm.at[wid], cidx_v)
    pltpu.sync_copy(x_hbm.at[pl.ds(s * _SLAB, _SLAB)],
                    x_sh.at[pl.ds(s * _SLAB, _SLAB)])
    plsc.subcore_barrier()

    lane = lax.iota(jnp.int32, 16)

    def group(g, base, rbuf, cbuf):
        e0 = g * 16
        for e16 in range(16):
            e = e0 + e16
            acc = rbuf[e, pl.ds(0, 16)] * cbuf[e, pl.ds(0, 16)]
            for q in range(1, 8):
                acc = acc + rbuf[e, pl.ds(q * 16, 16)] * cbuf[e, pl.ds(q * 16, 16)]
            pbuf[e16, :] = acc
        cols = jnp.zeros((16,), jnp.int32)
        tot = plsc.load_gather(pbuf, [lane, cols])
        for q in range(1, 16):
            cols = cols + 1
            tot = tot + plsc.load_gather(pbuf, [lane, cols])
        sig = 1.0 / (1.0 + jnp.exp(-tot))
        sbuf[pl.ds(base + g * 16, 16)] = sig

    def body(jj, carry):
        for h in range(2):
            pltpu.sync_copy(x_sh.at[ridx_v.at[jj, pl.ds(h * _KS, _KS)]], rbuf)
            pltpu.sync_copy(x_sh.at[cidx_v.at[jj, pl.ds(h * _KS, _KS)]], cbuf)
            base = jj * _K + h * _KS

            def gbody(g, carry2):
                group(g, base, rbuf, cbuf)
                return carry2

            lax.fori_loop(0, _KS // 16, gbody, 0)
        return carry

    lax.fori_loop(0, _CHUNKS, body, 0)
    pltpu.sync_copy(sbuf, out_hbm.at[pl.ds(wid * _EPT, _EPT)])


_score_call = functools.partial(
    pl.kernel,
    out_type=jax.ShapeDtypeStruct((_E_PAD,), jnp.float32),
    mesh=_sc_mesh,
    scratch_types=[
        pltpu.VMEM((_CHUNKS, _K), jnp.int32),
        pltpu.VMEM((_CHUNKS, _K), jnp.int32),
        pltpu.VMEM((_KS, _H), jnp.float32),
        pltpu.VMEM((_KS, _H), jnp.float32),
        pltpu.VMEM((_EPT,), jnp.float32),
        pltpu.VMEM((16, 16), jnp.float32),
        pltpu.VMEM_SHARED((_N_STAGE, _H), jnp.float32),
    ],
    compiler_params=_sc_params,
)(_score_body)


def _bn(x, g, b):
    m = jnp.mean(x, axis=0, keepdims=True)
    v = jnp.mean((x - m) ** 2, axis=0, keepdims=True)
    return (x - m) * lax.rsqrt(v + _EPS) * g + b


def _tc0_body(x_ref, gf_ref, bfeat_ref, wf_ref, bf_ref, degs_ref,
              g0_ref, b0_ref, wc0_ref, hs_ref, dis_ref):
    xb = _bn(x_ref[...], gf_ref[...], bfeat_ref[...])
    x1 = jnp.maximum(
        jnp.dot(xb, wf_ref[...], preferred_element_type=jnp.float32) + bf_ref[...],
        0.0)
    deg = degs_ref[0] + degs_ref[1] + 1.0
    dis = lax.rsqrt(deg)[: _N]
    xb1 = _bn(x1, g0_ref[...], b0_ref[...])
    hs_ref[...] = dis * jnp.dot(xb1, wc0_ref[...],
                                preferred_element_type=jnp.float32)
    dis_ref[...] = dis


_tc0_call = pl.pallas_call(
    _tc0_body,
    out_shape=[
        jax.ShapeDtypeStruct((_N, _H), jnp.float32),
        jax.ShapeDtypeStruct((_N, 1), jnp.float32),
    ],
)


def _tc_mid_body(agg_ref, hs_ref, dis_ref, bc_ref, g_ref, b_ref, wc_ref, out_ref):
    agg = agg_ref[0, : _N] + agg_ref[1, : _N]
    x2 = jnp.maximum(dis_ref[...] * (agg + hs_ref[...]) + bc_ref[...], 0.0)
    xb = _bn(x2, g_ref[...], b_ref[...])
    out_ref[...] = dis_ref[...] * jnp.dot(
        xb, wc_ref[...], preferred_element_type=jnp.float32)


_tc_mid_call = pl.pallas_call(
    _tc_mid_body,
    out_shape=jax.ShapeDtypeStruct((_N, _H), jnp.float32),
)


def _tc_last_body(agg_ref, hs_ref, dis_ref, bc_ref, out_ref):
    agg = agg_ref[0, : _N] + agg_ref[1, : _N]
    out_ref[: _N] = dis_ref[...] * (agg + hs_ref[...]) + bc_ref[...]


_tc_last_call = pl.pallas_call(
    _tc_last_body,
    out_shape=jax.ShapeDtypeStruct((_N_STAGE, _H), jnp.float32),
)


def kernel(x, edge_index, bn_feat_g, bn_feat_b, Wf, bf, bn_g, bn_b, Wc, bc):
    row = edge_index[0]
    col = edge_index[1]
    e = row.shape[0]
    pad = _E_PAD - e
    zpad = jnp.zeros((pad,), jnp.int32)
    tpad = jnp.full((pad,), _N, jnp.int32)
    row0f = jnp.concatenate([row, zpad])
    col0f = jnp.concatenate([col, zpad])
    row0p = row0f.reshape(_NW, _PNBLK, _PBLK, _K)
    colNp = jnp.concatenate([col, tpad]).reshape(_NW, _CHUNKS, _K)
    rowN = jnp.concatenate([row, tpad]).reshape(_NW, _CHUNKS, _K)
    row0s = row0f.reshape(_NW, _CHUNKS, _K)
    col0s = col0f.reshape(_NW, _CHUNKS, _K)
    zeros1 = jnp.zeros((_RPT,), jnp.float32)
    zeros2 = jnp.zeros((_RPT, _H), jnp.float32)
    ones_k = jnp.ones((_K,), jnp.float32)

    gf = bn_feat_g.reshape(1, _H)
    bfeat = bn_feat_b.reshape(1, _H)
    bf2 = bf.reshape(1, _H)

    degs = _deg_call(rowN, ones_k, zeros1)
    hs0, dis = _tc0_call(x, gf, bfeat, Wf, bf2, degs.reshape(_NC, _N_PAD, 1),
                         bn_g[0].reshape(1, _H), bn_b[0].reshape(1, _H), Wc[0])
    agg0 = _prop_call(hs0, row0p, colNp, zeros2)
    hs1 = _tc_mid_call(agg0, hs0, dis, bc[0].reshape(1, _H),
                       bn_g[1].reshape(1, _H), bn_b[1].reshape(1, _H), Wc[1])
    agg1 = _prop_call(hs1, row0p, colNp, zeros2)
    hs2 = _tc_mid_call(agg1, hs1, dis, bc[1].reshape(1, _H),
                       bn_g[2].reshape(1, _H), bn_b[2].reshape(1, _H), Wc[2])
    agg2 = _prop_call(hs2, row0p, colNp, zeros2)
    x4 = _tc_last_call(agg2, hs2, dis, bc[2].reshape(1, _H))
    scores = _score_call(x4, row0s, col0s)
    return scores[:e]
